# baseline (speedup 1.0000x reference)
import functools

import jax
import jax.numpy as jnp
from jax import lax
from jax.experimental import pallas as pl
from jax.experimental.pallas import tpu as pltpu
from jax.experimental.pallas import tpu_sc as plsc

N_NODES = 10000
N_EDGES = 320000
D = 128

NC = 2
NS = 16
NW = NC * NS
CHUNK = 128
EDGES_PER_W = N_EDGES // NW
SB = 8
NSB = 10
NCHUNK = SB * NSB
EDGES_PAD = NCHUNK * CHUNK
ACC_ROWS = 10112
ZROWS = ACC_ROWS // NS


def _mm_body(h_ref, w_ref, o_ref):
    o_ref[...] = jnp.dot(h_ref[...], w_ref[...],
                         preferred_element_type=jnp.float32)


def _matmul(h, W):
    return pl.pallas_call(
        _mm_body,
        grid=(10,),
        in_specs=[
            pl.BlockSpec((N_NODES // 10, D), lambda i: (i, 0)),
            pl.BlockSpec((D, D), lambda i: (0, 0)),
        ],
        out_specs=pl.BlockSpec((N_NODES // 10, D), lambda i: (i, 0)),
        out_shape=jax.ShapeDtypeStruct((N_NODES, D), jnp.float32),
    )(h, W)


def _scatter_body(h2_hbm, src_hbm, dst_hbm, zeros_hbm, parts_hbm,
                  acc, src_v, dst_v, rows0, rows1,
                  semg0, semg1, sems0, sems1, semi):
    c = lax.axis_index("c")
    s = lax.axis_index("s")
    wid = c * NS + s

    pltpu.sync_copy(zeros_hbm, acc.at[pl.ds(s * ZROWS, ZROWS)])

    plsc.subcore_barrier()

    rows = (rows0, rows1)
    semg = (semg0, semg1)
    sems = (sems0, sems1)

    def g_issue(p, k, b):
        pltpu.async_copy(h2_hbm.at[src_v.at[p, k]], rows[b], semg[b])

    def g_wait(p, k, b):
        pltpu.make_async_copy(h2_hbm.at[src_v.at[p, k]], rows[b],
                              semg[b]).wait()

    def s_issue(p, k, b):
        pltpu.async_copy(rows[b], acc.at[dst_v.at[p, k]], sems[b], add=True)

    def s_wait(p, k, b):
        pltpu.make_async_copy(rows[b], acc.at[dst_v.at[p, k]],
                              sems[b]).wait()

    pltpu.sync_copy(src_hbm.at[wid, 0], src_v.at[0])
    pltpu.sync_copy(dst_hbm.at[wid, 0], dst_v.at[0])
    g_issue(0, 0, 0)

    def sb_loop(sb, carry):
        p = sb % 2
        q = 1 - p

        for k in range(SB):
            b = k % 2
            g_wait(p, k, b)
            s_issue(p, k, b)
            if k == 0:
                @pl.when(sb > 0)
                def _():
                    s_wait(p, 0, 1 - b)

                @pl.when(sb < NSB - 1)
                def _():
                    pltpu.async_copy(src_hbm.at[wid, sb + 1],
                                     src_v.at[q], semi)
                    pltpu.async_copy(dst_hbm.at[wid, sb + 1],
                                     dst_v.at[q], semi)

                g_issue(p, k + 1, 1 - b)
            elif k < SB - 1:
                s_wait(p, k - 1, 1 - b)
                g_issue(p, k + 1, 1 - b)
            else:

                @pl.when(sb < NSB - 1)
                def _():
                    s_wait(p, k - 1, 1 - b)
                    pltpu.make_async_copy(src_hbm.at[wid, sb + 1],
                                          src_v.at[q], semi).wait()
                    pltpu.make_async_copy(dst_hbm.at[wid, sb + 1],
                                          dst_v.at[q], semi).wait()
                    g_issue(q, 0, 1 - b)

        return carry

    lax.fori_loop(0, NSB, sb_loop, 0)

    s_wait(1, SB - 2, 0)
    s_wait(1, SB - 1, 1)

    plsc.subcore_barrier()

    pltpu.sync_copy(acc.at[pl.ds(s * ZROWS, ZROWS)],
                    parts_hbm.at[c, pl.ds(s * ZROWS, ZROWS)])


def _message_pass(h2, src_p, dst_p, zeros):
    mesh = plsc.VectorSubcoreMesh(core_axis_name="c", subcore_axis_name="s")
    k = pl.kernel(
        _scatter_body,
        out_type=jax.ShapeDtypeStruct((NC, ACC_ROWS, D), jnp.float32),
        mesh=mesh,
        scratch_types=[
            pltpu.VMEM_SHARED((ACC_ROWS, D), jnp.float32),
            pltpu.VMEM((2, SB, CHUNK), jnp.int32),
            pltpu.VMEM((2, SB, CHUNK), jnp.int32),
            pltpu.VMEM((CHUNK, D), jnp.float32),
            pltpu.VMEM((CHUNK, D), jnp.float32),
            pltpu.SemaphoreType.DMA,
            pltpu.SemaphoreType.DMA,
            pltpu.SemaphoreType.DMA,
            pltpu.SemaphoreType.DMA,
            pltpu.SemaphoreType.DMA,
        ],
    )
    return k(h2, src_p, dst_p, zeros)


def _fin_body(p_ref, b_ref, o_ref):
    o_ref[...] = jnp.maximum(p_ref[0] + p_ref[1] + b_ref[...], 0.0)


def _finalize(parts, b2):
    return pl.pallas_call(
        _fin_body,
        grid=(10,),
        in_specs=[
            pl.BlockSpec((NC, N_NODES // 10, D), lambda i: (0, i, 0)),
            pl.BlockSpec((1, D), lambda i: (0, 0)),
        ],
        out_specs=pl.BlockSpec((N_NODES // 10, D), lambda i: (i, 0)),
        out_shape=jax.ShapeDtypeStruct((N_NODES, D), jnp.float32),
    )(parts, b2)


@jax.jit
def kernel(edge_index, h, W, b):
    src = edge_index[0].reshape(NW, EDGES_PER_W)
    dst = edge_index[1].reshape(NW, EDGES_PER_W)
    pad = EDGES_PAD - EDGES_PER_W
    src_p = jnp.pad(src, ((0, 0), (0, pad))).reshape(NW, NSB, SB, CHUNK)
    dst_p = jnp.pad(dst, ((0, 0), (0, pad)),
                    constant_values=N_NODES).reshape(NW, NSB, SB, CHUNK)
    zeros = jnp.zeros((ZROWS, D), jnp.float32)

    h2 = _matmul(h, W)
    parts = _message_pass(h2, src_p, dst_p, zeros)
    return _finalize(parts, b.reshape(1, D))


# speedup vs baseline: 1.3140x; 1.3140x over previous
import jax
import jax.numpy as jnp
from jax import lax
from jax.experimental import pallas as pl
from jax.experimental.pallas import tpu as pltpu
from jax.experimental.pallas import tpu_sc as plsc

N_NODES = 10000
N_EDGES = 320000
D = 128

NC = 2
NS = 16
NW = NC * NS
CHUNK = 128
EDGES_PER_W = N_EDGES // NW
NCHUNK = (EDGES_PER_W + CHUNK - 1) // CHUNK
EDGES_PAD = NCHUNK * CHUNK
ACC_ROWS = 10112
ZROWS = ACC_ROWS // NS


def _mm_body(h_ref, w_ref, o_ref):
    o_ref[...] = jnp.dot(h_ref[...], w_ref[...],
                         preferred_element_type=jnp.float32)


def _matmul(h, W):
    return pl.pallas_call(
        _mm_body,
        grid=(10,),
        in_specs=[
            pl.BlockSpec((N_NODES // 10, D), lambda i: (i, 0)),
            pl.BlockSpec((D, D), lambda i: (0, 0)),
        ],
        out_specs=pl.BlockSpec((N_NODES // 10, D), lambda i: (i, 0)),
        out_shape=jax.ShapeDtypeStruct((N_NODES, D), jnp.float32),
    )(h, W)


def _scatter_body(h2_hbm, src_hbm, dst_hbm, zeros_hbm, parts_hbm,
                  acc, src_v, dst_v, rows_v, sem):
    c = lax.axis_index("c")
    s = lax.axis_index("s")
    wid = c * NS + s

    pltpu.sync_copy(zeros_hbm, acc.at[pl.ds(s * ZROWS, ZROWS)])

    pltpu.sync_copy(src_hbm.at[wid], src_v)
    pltpu.sync_copy(dst_hbm.at[wid], dst_v)

    plsc.subcore_barrier()

    def step(j, carry):
        pltpu.async_copy(h2_hbm.at[src_v.at[j]], rows_v, sem).wait()
        pltpu.sync_copy(rows_v, acc.at[dst_v.at[j]], add=True)
        return carry

    lax.fori_loop(0, NCHUNK, step, 0)

    plsc.subcore_barrier()

    pltpu.sync_copy(acc.at[pl.ds(s * ZROWS, ZROWS)],
                    parts_hbm.at[c, pl.ds(s * ZROWS, ZROWS)])


def _message_pass(h2, src_p, dst_p, zeros):
    mesh = plsc.VectorSubcoreMesh(core_axis_name="c", subcore_axis_name="s")
    k = pl.kernel(
        _scatter_body,
        out_type=jax.ShapeDtypeStruct((NC, ACC_ROWS, D), jnp.float32),
        mesh=mesh,
        scratch_types=[
            pltpu.VMEM_SHARED((ACC_ROWS, D), jnp.float32),
            pltpu.VMEM((NCHUNK, CHUNK), jnp.int32),
            pltpu.VMEM((NCHUNK, CHUNK), jnp.int32),
            pltpu.VMEM((CHUNK, D), jnp.float32),
            pltpu.SemaphoreType.DMA,
        ],
    )
    return k(h2, src_p, dst_p, zeros)


def _fin_body(p_ref, b_ref, o_ref):
    o_ref[...] = jnp.maximum(p_ref[0] + p_ref[1] + b_ref[...], 0.0)


def _finalize(parts, b2):
    return pl.pallas_call(
        _fin_body,
        grid=(10,),
        in_specs=[
            pl.BlockSpec((NC, N_NODES // 10, D), lambda i: (0, i, 0)),
            pl.BlockSpec((1, D), lambda i: (0, 0)),
        ],
        out_specs=pl.BlockSpec((N_NODES // 10, D), lambda i: (i, 0)),
        out_shape=jax.ShapeDtypeStruct((N_NODES, D), jnp.float32),
    )(parts, b2)


@jax.jit
def kernel(edge_index, h, W, b):
    src = edge_index[0].reshape(NW, EDGES_PER_W)
    dst = edge_index[1].reshape(NW, EDGES_PER_W)
    pad = EDGES_PAD - EDGES_PER_W
    src_p = jnp.pad(src, ((0, 0), (0, pad))).reshape(NW, NCHUNK, CHUNK)
    dst_p = jnp.pad(dst, ((0, 0), (0, pad)),
                    constant_values=N_NODES).reshape(NW, NCHUNK, CHUNK)
    zeros = jnp.zeros((ZROWS, D), jnp.float32)

    h2 = _matmul(h, W)
    parts = _message_pass(h2, src_p, dst_p, zeros)
    return _finalize(parts, b.reshape(1, D))
